# Initial kernel scaffold; baseline (speedup 1.0000x reference)
import jax
import jax.numpy as jnp
from jax.experimental import pallas as pl


def kernel(x, Wr1, br1, Wr2, br2, W1, b1, W2, b2):
    raise NotImplementedError("write your pallas kernel here")


# speedup vs baseline: 1.8913x; 1.8913x over previous
import functools

import jax
import jax.numpy as jnp
from jax import lax
from jax.experimental import pallas as pl
from jax.experimental.pallas import tpu as pltpu
from jax.experimental.pallas import tpu_sc as plsc

N = 2048
D = 1024
DR = 256
E = 8
F = 2048
O = 1024
B = 128
NB = N // B
T = NB + E - 1
EP = 128

NC, NS = 2, 16
NW = NC * NS
CHUNK = N // NW


def _router_body(x_ref, wr1_ref, br1_ref, wr2_ref, br2_ref,
                 pos_ref, w2d_ref, meta_ref):
    x = x_ref[...]
    h = jnp.maximum(x @ wr1_ref[...] + br1_ref[...], 0.0)
    logits = h @ wr2_ref[...] + br2_ref[...]
    m = jnp.max(logits, axis=1, keepdims=True)
    p = jnp.exp(logits - m)
    prob = p / jnp.sum(p, axis=1, keepdims=True)
    lane = lax.broadcasted_iota(jnp.int32, (N, EP), 1)
    m1 = jnp.max(prob, axis=1, keepdims=True)
    i1 = jnp.min(jnp.where(prob == m1, lane, EP), axis=1, keepdims=True)
    probx = jnp.where(lane == i1, -1.0, prob)
    m2 = jnp.max(probx, axis=1, keepdims=True)
    i2 = jnp.min(jnp.where(probx == m2, lane, EP), axis=1, keepdims=True)
    t = jnp.exp(m2 - m1)
    nw1 = 1.0 / (1.0 + t)
    nw2 = t / (1.0 + t)
    estar = jnp.maximum(i1, i2)
    w = jnp.where(i1 > i2, nw1, nw2)

    onehot = (lane == estar).astype(jnp.float32)
    r_i = lax.broadcasted_iota(jnp.int32, (N, N), 0)
    c_i = lax.broadcasted_iota(jnp.int32, (N, N), 1)
    Lm = (c_i < r_i).astype(jnp.float32)
    rank = lax.dot(Lm, onehot, precision=lax.Precision.HIGHEST)
    counts = jnp.sum(onehot, axis=0, keepdims=True)
    r2 = lax.broadcasted_iota(jnp.int32, (EP, EP), 0)
    c2 = lax.broadcasted_iota(jnp.int32, (EP, EP), 1)
    Um = (r2 < c2).astype(jnp.float32)
    offs = lax.dot(counts, Um, precision=lax.Precision.HIGHEST)
    pos = jnp.sum(onehot * (offs + rank), axis=1)
    pos_ref[...] = pos.astype(jnp.int32).reshape(16, 128)
    w2d_ref[...] = jnp.broadcast_to(w, (N, 128))

    offs_i = offs.astype(jnp.int32)
    cnts_i = counts.astype(jnp.int32)
    kio = lax.broadcasted_iota(jnp.int32, (1, 128), 1)
    kb = kio // E
    ke = kio % E
    o_k = jnp.zeros((1, 128), jnp.int32)
    c_k = jnp.zeros((1, 128), jnp.int32)
    for e in range(E):
        oe = offs_i[0:1, e:e + 1]
        ce = cnts_i[0:1, e:e + 1]
        sel = ke == e
        o_k = jnp.where(sel, oe, o_k)
        c_k = jnp.where(sel, ce, c_k)
    lo = kb * B
    hi = lo + B
    M = (c_k > 0) & (o_k < hi) & ((o_k + c_k) > lo)
    Mf = M.astype(jnp.float32)
    tri = (lax.broadcasted_iota(jnp.int32, (128, 128), 0)
           <= lax.broadcasted_iota(jnp.int32, (128, 128), 1)).astype(jnp.float32)
    cs = lax.dot(Mf, tri, precision=lax.Precision.HIGHEST).astype(jnp.int32)
    total = jnp.max(cs)
    srow = lax.broadcasted_iota(jnp.int32, (128, 128), 0)
    A = (cs == (srow + 1)) & M
    kio_col = lax.broadcasted_iota(jnp.int32, (128, 128), 1)
    j = jnp.sum(jnp.where(A, kio_col, 0), axis=1, keepdims=True)
    jlast = jnp.max(jnp.where((cs == total) & M, kio, 0))
    svec = lax.broadcasted_iota(jnp.int32, (128, 1), 0)
    j = jnp.where(svec < total, j, jlast)
    bid = (j // E).reshape(1, 128)
    eid = (j % E).reshape(1, 128)
    row = lax.broadcasted_iota(jnp.int32, (8, 128), 0)
    ends_i = offs_i + cnts_i
    meta = jnp.where(row == 0, jnp.broadcast_to(bid, (8, 128)), 0)
    meta = jnp.where(row == 1, jnp.broadcast_to(eid, (8, 128)), meta)
    meta = jnp.where(row == 2, jnp.broadcast_to(offs_i, (8, 128)), meta)
    meta = jnp.where(row == 3, jnp.broadcast_to(ends_i, (8, 128)), meta)
    meta_ref[...] = meta


def _router(x, Wr1, br1, Wr2, br2):
    Wr2p = jnp.pad(Wr2, ((0, 0), (0, EP - E)))
    br2p = jnp.pad(br2, (0, EP - E), constant_values=-1e30).reshape(1, EP)
    return pl.pallas_call(
        _router_body,
        out_shape=(
            jax.ShapeDtypeStruct((16, 128), jnp.int32),
            jax.ShapeDtypeStruct((N, 128), jnp.float32),
            jax.ShapeDtypeStruct((8, 128), jnp.int32),
        ),
    )(x, Wr1, br1.reshape(1, DR), Wr2p, br2p)


def _dispatch_body(pos_hbm, x_hbm, w2d_hbm, xs_hbm, ws_hbm,
                   idx_v, rows_v, w_v, sem):
    wid = lax.axis_index("s") * NC + lax.axis_index("c")
    base = wid * CHUNK
    pltpu.sync_copy(pos_hbm.at[pl.ds(base, CHUNK)], idx_v)
    pltpu.sync_copy(x_hbm.at[pl.ds(base, CHUNK)], rows_v)
    pltpu.async_copy(rows_v, xs_hbm.at[idx_v], sem).wait()
    pltpu.sync_copy(w2d_hbm.at[pl.ds(base, CHUNK)], w_v)
    pltpu.async_copy(w_v, ws_hbm.at[idx_v], sem).wait()


_dispatch = pl.kernel(
    _dispatch_body,
    out_type=(
        jax.ShapeDtypeStruct((N, D), jnp.float32),
        jax.ShapeDtypeStruct((N, 128), jnp.float32),
    ),
    mesh=plsc.VectorSubcoreMesh(core_axis_name="c", subcore_axis_name="s"),
    scratch_types=[
        pltpu.VMEM((CHUNK,), jnp.int32),
        pltpu.VMEM((CHUNK, D), jnp.float32),
        pltpu.VMEM((CHUNK, 128), jnp.float32),
        pltpu.SemaphoreType.DMA,
    ],
)


def _mm_body(meta_ref, xs_ref, w1_ref, b1_ref, w2_ref, b2_ref, ws_ref, out_ref):
    t = pl.program_id(0)
    e = meta_ref[128 + t]
    bid = meta_ref[t]
    start = meta_ref[256 + e]
    end = meta_ref[384 + e]
    x = xs_ref[...]
    h = jnp.maximum(x @ w1_ref[0] + b1_ref[0], 0.0)
    o = h @ w2_ref[0] + b2_ref[0]
    o = o * ws_ref[:, 0:1]
    rows = bid * B + lax.broadcasted_iota(jnp.int32, (B, 1), 0)
    owned = (rows >= start) & (rows < end)
    out_ref[...] = jnp.where(owned, o, out_ref[...])


def _grouped_mm(meta_flat, xs, W1, b1, W2, b2, ws):
    gs = pltpu.PrefetchScalarGridSpec(
        num_scalar_prefetch=1,
        grid=(T,),
        in_specs=[
            pl.BlockSpec((B, D), lambda t, m: (m[t], 0)),
            pl.BlockSpec((1, D, F), lambda t, m: (m[128 + t], 0, 0)),
            pl.BlockSpec((1, 1, F), lambda t, m: (m[128 + t], 0, 0)),
            pl.BlockSpec((1, F, O), lambda t, m: (m[128 + t], 0, 0)),
            pl.BlockSpec((1, 1, O), lambda t, m: (m[128 + t], 0, 0)),
            pl.BlockSpec((B, 128), lambda t, m: (m[t], 0)),
        ],
        out_specs=pl.BlockSpec((B, O), lambda t, m: (m[t], 0)),
    )
    return pl.pallas_call(
        _mm_body, grid_spec=gs,
        out_shape=jax.ShapeDtypeStruct((N, O), jnp.float32),
    )(meta_flat, xs, W1, b1.reshape(E, 1, F), W2, b2.reshape(E, 1, O), ws)


def _combine_body(pos_hbm, outs_hbm, out_hbm, idx_v, rows_v, sem):
    wid = lax.axis_index("s") * NC + lax.axis_index("c")
    base = wid * CHUNK
    pltpu.sync_copy(pos_hbm.at[pl.ds(base, CHUNK)], idx_v)
    pltpu.async_copy(outs_hbm.at[idx_v], rows_v, sem).wait()
    pltpu.sync_copy(rows_v, out_hbm.at[pl.ds(base, CHUNK)])


_combine = pl.kernel(
    _combine_body,
    out_type=jax.ShapeDtypeStruct((N, O), jnp.float32),
    mesh=plsc.VectorSubcoreMesh(core_axis_name="c", subcore_axis_name="s"),
    scratch_types=[
        pltpu.VMEM((CHUNK,), jnp.int32),
        pltpu.VMEM((CHUNK, O), jnp.float32),
        pltpu.SemaphoreType.DMA,
    ],
)


@jax.jit
def kernel(x, Wr1, br1, Wr2, br2, W1, b1, W2, b2):
    pos2d, w2d, meta = _router(x, Wr1, br1, Wr2, br2)
    pos = pos2d.reshape(-1)
    meta_flat = meta.reshape(-1)
    xs, ws = _dispatch(pos, x, w2d)
    outs = _grouped_mm(meta_flat, xs, W1, b1, W2, b2, ws)
    return _combine(pos, outs)


# speedup vs baseline: 2.1044x; 1.1127x over previous
import functools

import jax
import jax.numpy as jnp
from jax import lax
from jax.experimental import pallas as pl
from jax.experimental.pallas import tpu as pltpu
from jax.experimental.pallas import tpu_sc as plsc

N = 2048
D = 1024
DR = 256
E = 8
F = 2048
O = 1024
B = 128
NB = N // B
T = NB + E - 1
EP = 128

NC, NS = 2, 16
NW = NC * NS
CHUNK = N // NW


def _router_body(x_ref, wr1_ref, br1_ref, wr2_ref, br2_ref,
                 pos_ref, w2d_ref, meta_ref):
    x = x_ref[...]
    h = jnp.maximum(x @ wr1_ref[...] + br1_ref[...], 0.0)
    logits = h @ wr2_ref[...] + br2_ref[...]
    m = jnp.max(logits, axis=1, keepdims=True)
    p = jnp.exp(logits - m)
    prob = p / jnp.sum(p, axis=1, keepdims=True)
    lane = lax.broadcasted_iota(jnp.int32, (N, EP), 1)
    m1 = jnp.max(prob, axis=1, keepdims=True)
    i1 = jnp.min(jnp.where(prob == m1, lane, EP), axis=1, keepdims=True)
    probx = jnp.where(lane == i1, -1.0, prob)
    m2 = jnp.max(probx, axis=1, keepdims=True)
    i2 = jnp.min(jnp.where(probx == m2, lane, EP), axis=1, keepdims=True)
    t = jnp.exp(m2 - m1)
    nw1 = 1.0 / (1.0 + t)
    nw2 = t / (1.0 + t)
    estar = jnp.maximum(i1, i2)
    w = jnp.where(i1 > i2, nw1, nw2)

    onehot = (lane == estar).astype(jnp.float32)
    s = onehot
    k = 1
    while k < N:
        s = s + jnp.concatenate(
            [jnp.zeros((k, EP), jnp.float32), s[:-k]], axis=0)
        k *= 2
    rank = s - onehot
    counts = jnp.sum(onehot, axis=0, keepdims=True)
    r2 = lax.broadcasted_iota(jnp.int32, (EP, EP), 0)
    c2 = lax.broadcasted_iota(jnp.int32, (EP, EP), 1)
    Um = (r2 < c2).astype(jnp.float32)
    offs = lax.dot(counts, Um, precision=lax.Precision.HIGHEST)
    pos = jnp.sum(onehot * (offs + rank), axis=1)
    pos_ref[...] = pos.astype(jnp.int32).reshape(16, 128)
    w2d_ref[...] = jnp.broadcast_to(w, (N, 128))

    offs_i = offs.astype(jnp.int32)
    cnts_i = counts.astype(jnp.int32)
    kio = lax.broadcasted_iota(jnp.int32, (1, 128), 1)
    kb = kio // E
    ke = kio % E
    o_k = jnp.zeros((1, 128), jnp.int32)
    c_k = jnp.zeros((1, 128), jnp.int32)
    for e in range(E):
        oe = offs_i[0:1, e:e + 1]
        ce = cnts_i[0:1, e:e + 1]
        sel = ke == e
        o_k = jnp.where(sel, oe, o_k)
        c_k = jnp.where(sel, ce, c_k)
    lo = kb * B
    hi = lo + B
    M = (c_k > 0) & (o_k < hi) & ((o_k + c_k) > lo)
    Mf = M.astype(jnp.float32)
    tri = (lax.broadcasted_iota(jnp.int32, (128, 128), 0)
           <= lax.broadcasted_iota(jnp.int32, (128, 128), 1)).astype(jnp.float32)
    cs = lax.dot(Mf, tri, precision=lax.Precision.HIGHEST).astype(jnp.int32)
    total = jnp.max(cs)
    srow = lax.broadcasted_iota(jnp.int32, (128, 128), 0)
    A = (cs == (srow + 1)) & M
    kio_col = lax.broadcasted_iota(jnp.int32, (128, 128), 1)
    j = jnp.sum(jnp.where(A, kio_col, 0), axis=1, keepdims=True)
    jlast = jnp.max(jnp.where((cs == total) & M, kio, 0))
    svec = lax.broadcasted_iota(jnp.int32, (128, 1), 0)
    j = jnp.where(svec < total, j, jlast)
    bid = (j // E).reshape(1, 128)
    eid = (j % E).reshape(1, 128)
    row = lax.broadcasted_iota(jnp.int32, (8, 128), 0)
    ends_i = offs_i + cnts_i
    meta = jnp.where(row == 0, jnp.broadcast_to(bid, (8, 128)), 0)
    meta = jnp.where(row == 1, jnp.broadcast_to(eid, (8, 128)), meta)
    meta = jnp.where(row == 2, jnp.broadcast_to(offs_i, (8, 128)), meta)
    meta = jnp.where(row == 3, jnp.broadcast_to(ends_i, (8, 128)), meta)
    meta_ref[...] = meta


def _router(x, Wr1, br1, Wr2, br2):
    Wr2p = jnp.pad(Wr2, ((0, 0), (0, EP - E)))
    br2p = jnp.pad(br2, (0, EP - E), constant_values=-1e30).reshape(1, EP)
    return pl.pallas_call(
        _router_body,
        out_shape=(
            jax.ShapeDtypeStruct((16, 128), jnp.int32),
            jax.ShapeDtypeStruct((N, 128), jnp.float32),
            jax.ShapeDtypeStruct((8, 128), jnp.int32),
        ),
    )(x, Wr1, br1.reshape(1, DR), Wr2p, br2p)


def _dispatch_body(pos_hbm, x_hbm, w2d_hbm, xs_hbm, ws_hbm,
                   idx_v, rows_v, w_v, sem):
    wid = lax.axis_index("s") * NC + lax.axis_index("c")
    base = wid * CHUNK
    pltpu.sync_copy(pos_hbm.at[pl.ds(base, CHUNK)], idx_v)
    pltpu.sync_copy(x_hbm.at[pl.ds(base, CHUNK)], rows_v)
    pltpu.async_copy(rows_v, xs_hbm.at[idx_v], sem).wait()
    pltpu.sync_copy(w2d_hbm.at[pl.ds(base, CHUNK)], w_v)
    pltpu.async_copy(w_v, ws_hbm.at[idx_v], sem).wait()


_dispatch = pl.kernel(
    _dispatch_body,
    out_type=(
        jax.ShapeDtypeStruct((N, D), jnp.float32),
        jax.ShapeDtypeStruct((N, 128), jnp.float32),
    ),
    mesh=plsc.VectorSubcoreMesh(core_axis_name="c", subcore_axis_name="s"),
    scratch_types=[
        pltpu.VMEM((CHUNK,), jnp.int32),
        pltpu.VMEM((CHUNK, D), jnp.float32),
        pltpu.VMEM((CHUNK, 128), jnp.float32),
        pltpu.SemaphoreType.DMA,
    ],
)


def _mm_body(meta_ref, xs_ref, w1_ref, b1_ref, w2_ref, b2_ref, ws_ref, out_ref):
    t = pl.program_id(0)
    e = meta_ref[128 + t]
    bid = meta_ref[t]
    start = meta_ref[256 + e]
    end = meta_ref[384 + e]
    x = xs_ref[...].astype(jnp.bfloat16)
    h = jnp.maximum(
        jnp.dot(x, w1_ref[0].astype(jnp.bfloat16),
                preferred_element_type=jnp.float32) + b1_ref[0], 0.0)
    o = jnp.dot(h.astype(jnp.bfloat16), w2_ref[0].astype(jnp.bfloat16),
                preferred_element_type=jnp.float32) + b2_ref[0]
    o = o * ws_ref[:, 0:1]
    rows = bid * B + lax.broadcasted_iota(jnp.int32, (B, 1), 0)
    owned = (rows >= start) & (rows < end)
    out_ref[...] = jnp.where(owned, o, out_ref[...])


def _grouped_mm(meta_flat, xs, W1, b1, W2, b2, ws):
    gs = pltpu.PrefetchScalarGridSpec(
        num_scalar_prefetch=1,
        grid=(T,),
        in_specs=[
            pl.BlockSpec((B, D), lambda t, m: (m[t], 0)),
            pl.BlockSpec((1, D, F), lambda t, m: (m[128 + t], 0, 0)),
            pl.BlockSpec((1, 1, F), lambda t, m: (m[128 + t], 0, 0)),
            pl.BlockSpec((1, F, O), lambda t, m: (m[128 + t], 0, 0)),
            pl.BlockSpec((1, 1, O), lambda t, m: (m[128 + t], 0, 0)),
            pl.BlockSpec((B, 128), lambda t, m: (m[t], 0)),
        ],
        out_specs=pl.BlockSpec((B, O), lambda t, m: (m[t], 0)),
    )
    return pl.pallas_call(
        _mm_body, grid_spec=gs,
        out_shape=jax.ShapeDtypeStruct((N, O), jnp.float32),
    )(meta_flat, xs, W1, b1.reshape(E, 1, F), W2, b2.reshape(E, 1, O), ws)


def _combine_body(pos_hbm, outs_hbm, out_hbm, idx_v, rows_v, sem):
    wid = lax.axis_index("s") * NC + lax.axis_index("c")
    base = wid * CHUNK
    pltpu.sync_copy(pos_hbm.at[pl.ds(base, CHUNK)], idx_v)
    pltpu.async_copy(outs_hbm.at[idx_v], rows_v, sem).wait()
    pltpu.sync_copy(rows_v, out_hbm.at[pl.ds(base, CHUNK)])


_combine = pl.kernel(
    _combine_body,
    out_type=jax.ShapeDtypeStruct((N, O), jnp.float32),
    mesh=plsc.VectorSubcoreMesh(core_axis_name="c", subcore_axis_name="s"),
    scratch_types=[
        pltpu.VMEM((CHUNK,), jnp.int32),
        pltpu.VMEM((CHUNK, O), jnp.float32),
        pltpu.SemaphoreType.DMA,
    ],
)


@jax.jit
def kernel(x, Wr1, br1, Wr2, br2, W1, b1, W2, b2):
    pos2d, w2d, meta = _router(x, Wr1, br1, Wr2, br2)
    pos = pos2d.reshape(-1)
    meta_flat = meta.reshape(-1)
    xs, ws = _dispatch(pos, x, w2d)
    outs = _grouped_mm(meta_flat, xs, W1, b1, W2, b2, ws)
    return _combine(pos, outs)


# speedup vs baseline: 2.1730x; 1.0326x over previous
import jax
import jax.numpy as jnp
from jax import lax
from jax.experimental import pallas as pl
from jax.experimental.pallas import tpu as pltpu
from jax.experimental.pallas import tpu_sc as plsc

N = 2048
D = 1024
DR = 256
E = 8
F = 2048
O = 1024
B = 128
NB = N // B
T = NB + E - 1
WL = 128
XH = D // 2
DW = XH + WL

NC, NS = 2, 16
NW = NC * NS
CHUNK = N // NW


def _router_body(x_ref, wr1_ref, br1_ref, wr2_ref, br2_ref,
                 pos_ref, meta_ref, xw_ref):
    x = x_ref[...]
    h = jnp.maximum(x @ wr1_ref[...] + br1_ref[...], 0.0)
    logits = h @ wr2_ref[...] + br2_ref[...]
    m = jnp.max(logits, axis=1, keepdims=True)
    p = jnp.exp(logits - m)
    prob = p / jnp.sum(p, axis=1, keepdims=True)
    lane = lax.broadcasted_iota(jnp.int32, (N, E), 1)
    m1 = jnp.max(prob, axis=1, keepdims=True)
    i1 = jnp.min(jnp.where(prob == m1, lane, E), axis=1, keepdims=True)
    probx = jnp.where(lane == i1, -1.0, prob)
    m2 = jnp.max(probx, axis=1, keepdims=True)
    i2 = jnp.min(jnp.where(probx == m2, lane, E), axis=1, keepdims=True)
    t = jnp.exp(m2 - m1)
    nw1 = 1.0 / (1.0 + t)
    nw2 = t / (1.0 + t)
    estar = jnp.maximum(i1, i2)
    w = jnp.where(i1 > i2, nw1, nw2)

    onehot = (lane == estar).astype(jnp.float32)
    s = onehot
    k = 1
    while k < N:
        s = s + jnp.concatenate(
            [jnp.zeros((k, E), jnp.float32), s[:-k]], axis=0)
        k *= 2
    rank = s - onehot
    counts = jnp.sum(onehot, axis=0, keepdims=True)
    offs_list = [jnp.zeros((1, 1), jnp.float32)]
    run = jnp.zeros((1, 1), jnp.float32)
    for e in range(E - 1):
        run = run + counts[0:1, e:e + 1]
        offs_list.append(run)
    offs = jnp.concatenate(offs_list, axis=1)
    pos = jnp.sum(onehot * (offs + rank), axis=1)
    pos_ref[...] = pos.astype(jnp.int32).reshape(16, 128)

    ua = lax.bitcast_convert_type(x[:, :XH], jnp.uint32)
    ub = lax.bitcast_convert_type(x[:, XH:], jnp.uint32)
    packed_x = (ub & jnp.uint32(0xFFFF0000)) | (ua >> 16)
    wseg = lax.bitcast_convert_type(jnp.broadcast_to(w, (N, WL)), jnp.uint32)
    xw_ref[...] = jnp.concatenate([packed_x, wseg], axis=1)

    offs_i = offs.astype(jnp.int32)
    cnts_i = counts.astype(jnp.int32)
    kio = lax.broadcasted_iota(jnp.int32, (1, 128), 1)
    kb = kio // E
    ke = kio % E
    o_k = jnp.zeros((1, 128), jnp.int32)
    c_k = jnp.zeros((1, 128), jnp.int32)
    for e in range(E):
        oe = offs_i[0:1, e:e + 1]
        ce = cnts_i[0:1, e:e + 1]
        sel = ke == e
        o_k = jnp.where(sel, oe, o_k)
        c_k = jnp.where(sel, ce, c_k)
    lo = kb * B
    hi = lo + B
    M = (c_k > 0) & (o_k < hi) & ((o_k + c_k) > lo)
    Mf = M.astype(jnp.float32)
    tri = (lax.broadcasted_iota(jnp.int32, (128, 128), 0)
           <= lax.broadcasted_iota(jnp.int32, (128, 128), 1)).astype(jnp.float32)
    cs = lax.dot(Mf, tri, precision=lax.Precision.HIGHEST).astype(jnp.int32)
    total = jnp.max(cs)
    srow = lax.broadcasted_iota(jnp.int32, (128, 128), 0)
    A = (cs == (srow + 1)) & M
    kio_col = lax.broadcasted_iota(jnp.int32, (128, 128), 1)
    j = jnp.sum(jnp.where(A, kio_col, 0), axis=1, keepdims=True)
    jlast = jnp.max(jnp.where((cs == total) & M, kio, 0))
    svec = lax.broadcasted_iota(jnp.int32, (128, 1), 0)
    j = jnp.where(svec < total, j, jlast)
    bid = (j // E).reshape(1, 128)
    eid = (j % E).reshape(1, 128)
    row = lax.broadcasted_iota(jnp.int32, (8, 128), 0)
    pad = jnp.zeros((1, 128 - E), jnp.int32)
    offs_p = jnp.concatenate([offs_i, pad], axis=1)
    ends_p = jnp.concatenate([offs_i + cnts_i, pad], axis=1)
    meta = jnp.where(row == 0, jnp.broadcast_to(bid, (8, 128)), 0)
    meta = jnp.where(row == 1, jnp.broadcast_to(eid, (8, 128)), meta)
    meta = jnp.where(row == 2, jnp.broadcast_to(offs_p, (8, 128)), meta)
    meta = jnp.where(row == 3, jnp.broadcast_to(ends_p, (8, 128)), meta)
    meta_ref[...] = meta


def _router(x, Wr1, br1, Wr2, br2):
    return pl.pallas_call(
        _router_body,
        out_shape=(
            jax.ShapeDtypeStruct((16, 128), jnp.int32),
            jax.ShapeDtypeStruct((8, 128), jnp.int32),
            jax.ShapeDtypeStruct((N, DW), jnp.uint32),
        ),
    )(x, Wr1, br1.reshape(1, DR), Wr2, br2.reshape(1, E))


def _dispatch_body(pos_hbm, xw_hbm, xws_hbm, idx_v, rows_v, sem):
    wid = lax.axis_index("s") * NC + lax.axis_index("c")
    base = wid * CHUNK
    pltpu.sync_copy(pos_hbm.at[pl.ds(base, CHUNK)], idx_v)
    pltpu.sync_copy(xw_hbm.at[pl.ds(base, CHUNK)], rows_v)
    pltpu.async_copy(rows_v, xws_hbm.at[idx_v], sem).wait()


_dispatch = pl.kernel(
    _dispatch_body,
    out_type=jax.ShapeDtypeStruct((N, DW), jnp.uint32),
    mesh=plsc.VectorSubcoreMesh(core_axis_name="c", subcore_axis_name="s"),
    scratch_types=[
        pltpu.VMEM((CHUNK,), jnp.int32),
        pltpu.VMEM((CHUNK, DW), jnp.uint32),
        pltpu.SemaphoreType.DMA,
    ],
)


def _mm_body(meta_ref, xs_ref, w1_ref, b1_ref, w2_ref, b2_ref, out_ref):
    t = pl.program_id(0)
    e = meta_ref[128 + t]
    bid = meta_ref[t]
    start = meta_ref[256 + e]
    end = meta_ref[384 + e]
    xw = xs_ref[...]
    px = xw[:, :XH]
    xa = lax.bitcast_convert_type(px << 16, jnp.float32)
    xb = lax.bitcast_convert_type(px & jnp.uint32(0xFFFF0000), jnp.float32)
    x = jnp.concatenate([xa, xb], axis=1).astype(jnp.bfloat16)
    wcol = lax.bitcast_convert_type(xw[:, XH:XH + 1], jnp.float32)
    h = jnp.maximum(
        jnp.dot(x, w1_ref[0].astype(jnp.bfloat16),
                preferred_element_type=jnp.float32) + b1_ref[0], 0.0)
    o = jnp.dot(h.astype(jnp.bfloat16), w2_ref[0].astype(jnp.bfloat16),
                preferred_element_type=jnp.float32) + b2_ref[0]
    o = o * wcol
    rows = bid * B + lax.broadcasted_iota(jnp.int32, (B, 1), 0)
    owned = (rows >= start) & (rows < end)
    out_ref[...] = jnp.where(owned, o, out_ref[...])


def _grouped_mm(meta_flat, xws, W1, b1, W2, b2):
    gs = pltpu.PrefetchScalarGridSpec(
        num_scalar_prefetch=1,
        grid=(T,),
        in_specs=[
            pl.BlockSpec((B, DW), lambda t, m: (m[t], 0)),
            pl.BlockSpec((1, D, F), lambda t, m: (m[128 + t], 0, 0)),
            pl.BlockSpec((1, 1, F), lambda t, m: (m[128 + t], 0, 0)),
            pl.BlockSpec((1, F, O), lambda t, m: (m[128 + t], 0, 0)),
            pl.BlockSpec((1, 1, O), lambda t, m: (m[128 + t], 0, 0)),
        ],
        out_specs=pl.BlockSpec((B, O), lambda t, m: (m[t], 0)),
    )
    return pl.pallas_call(
        _mm_body, grid_spec=gs,
        out_shape=jax.ShapeDtypeStruct((N, O), jnp.float32),
    )(meta_flat, xws, W1, b1.reshape(E, 1, F), W2, b2.reshape(E, 1, O))


def _combine_body(pos_hbm, outs_hbm, out_hbm, idx_v, rows_v, sem):
    wid = lax.axis_index("s") * NC + lax.axis_index("c")
    base = wid * CHUNK
    pltpu.sync_copy(pos_hbm.at[pl.ds(base, CHUNK)], idx_v)
    pltpu.async_copy(outs_hbm.at[idx_v], rows_v, sem).wait()
    pltpu.sync_copy(rows_v, out_hbm.at[pl.ds(base, CHUNK)])


_combine = pl.kernel(
    _combine_body,
    out_type=jax.ShapeDtypeStruct((N, O), jnp.float32),
    mesh=plsc.VectorSubcoreMesh(core_axis_name="c", subcore_axis_name="s"),
    scratch_types=[
        pltpu.VMEM((CHUNK,), jnp.int32),
        pltpu.VMEM((CHUNK, O), jnp.float32),
        pltpu.SemaphoreType.DMA,
    ],
)


@jax.jit
def kernel(x, Wr1, br1, Wr2, br2, W1, b1, W2, b2):
    pos2d, meta, xw = _router(x, Wr1, br1, Wr2, br2)
    pos = pos2d.reshape(-1)
    xws = _dispatch(pos, xw)
    outs = _grouped_mm(meta.reshape(-1), xws, W1, b1, W2, b2)
    return _combine(pos, outs)


# speedup vs baseline: 2.2297x; 1.0261x over previous
import jax
import jax.numpy as jnp
from jax import lax
from jax.experimental import pallas as pl
from jax.experimental.pallas import tpu as pltpu
from jax.experimental.pallas import tpu_sc as plsc

N = 2048
D = 1024
DR = 256
E = 8
F = 2048
O = 1024
B = 128
NB = N // B
T = NB + E - 1
WL = 128
XH = D // 2
DW = XH + WL

NC, NS = 2, 16
NW = NC * NS
CHUNK = N // NW


def _router_body(x_ref, wr1_ref, br1_ref, wr2_ref, br2_ref,
                 pos_ref, meta_ref, xw_ref):
    x = x_ref[...]
    h = jnp.maximum(x @ wr1_ref[...] + br1_ref[...], 0.0)
    logits = h @ wr2_ref[...] + br2_ref[...]
    m = jnp.max(logits, axis=1, keepdims=True)
    p = jnp.exp(logits - m)
    prob = p / jnp.sum(p, axis=1, keepdims=True)
    lane = lax.broadcasted_iota(jnp.int32, (N, E), 1)
    m1 = jnp.max(prob, axis=1, keepdims=True)
    i1 = jnp.min(jnp.where(prob == m1, lane, E), axis=1, keepdims=True)
    probx = jnp.where(lane == i1, -1.0, prob)
    m2 = jnp.max(probx, axis=1, keepdims=True)
    i2 = jnp.min(jnp.where(probx == m2, lane, E), axis=1, keepdims=True)
    t = jnp.exp(m2 - m1)
    nw1 = 1.0 / (1.0 + t)
    nw2 = t / (1.0 + t)
    estar = jnp.maximum(i1, i2)
    w = jnp.where(i1 > i2, nw1, nw2)

    onehot = (lane == estar).astype(jnp.float32)
    s = onehot
    k = 1
    while k < N:
        s = s + jnp.concatenate(
            [jnp.zeros((k, E), jnp.float32), s[:-k]], axis=0)
        k *= 2
    rank = s - onehot
    counts = jnp.sum(onehot, axis=0, keepdims=True)
    offs_list = [jnp.zeros((1, 1), jnp.float32)]
    run = jnp.zeros((1, 1), jnp.float32)
    for e in range(E - 1):
        run = run + counts[0:1, e:e + 1]
        offs_list.append(run)
    offs = jnp.concatenate(offs_list, axis=1)
    pos = jnp.sum(onehot * (offs + rank), axis=1)
    pos_ref[...] = pos.astype(jnp.int32).reshape(16, 128)

    ua = lax.bitcast_convert_type(x[:, :XH], jnp.uint32)
    ub = lax.bitcast_convert_type(x[:, XH:], jnp.uint32)
    packed_x = (ub & jnp.uint32(0xFFFF0000)) | (ua >> 16)
    wseg = lax.bitcast_convert_type(jnp.broadcast_to(w, (N, WL)), jnp.uint32)
    xw_ref[...] = jnp.concatenate([packed_x, wseg], axis=1)

    offs_i = offs.astype(jnp.int32)
    cnts_i = counts.astype(jnp.int32)
    kio = lax.broadcasted_iota(jnp.int32, (1, 128), 1)
    kb = kio // E
    ke = kio % E
    o_k = jnp.zeros((1, 128), jnp.int32)
    c_k = jnp.zeros((1, 128), jnp.int32)
    for e in range(E):
        oe = offs_i[0:1, e:e + 1]
        ce = cnts_i[0:1, e:e + 1]
        sel = ke == e
        o_k = jnp.where(sel, oe, o_k)
        c_k = jnp.where(sel, ce, c_k)
    lo = kb * B
    hi = lo + B
    M = (c_k > 0) & (o_k < hi) & ((o_k + c_k) > lo)
    Mf = M.astype(jnp.float32)
    tri = (lax.broadcasted_iota(jnp.int32, (128, 128), 0)
           <= lax.broadcasted_iota(jnp.int32, (128, 128), 1)).astype(jnp.float32)
    cs = lax.dot(Mf, tri, precision=lax.Precision.HIGHEST).astype(jnp.int32)
    total = jnp.max(cs)
    srow = lax.broadcasted_iota(jnp.int32, (128, 128), 0)
    A = (cs == (srow + 1)) & M
    kio_col = lax.broadcasted_iota(jnp.int32, (128, 128), 1)
    j = jnp.sum(jnp.where(A, kio_col, 0), axis=1, keepdims=True)
    jlast = jnp.max(jnp.where((cs == total) & M, kio, 0))
    svec = lax.broadcasted_iota(jnp.int32, (128, 1), 0)
    j = jnp.where(svec < total, j, jlast)
    bid = (j // E).reshape(1, 128)
    eid = (j % E).reshape(1, 128)
    row = lax.broadcasted_iota(jnp.int32, (8, 128), 0)
    pad = jnp.zeros((1, 128 - E), jnp.int32)
    offs_p = jnp.concatenate([offs_i, pad], axis=1)
    ends_p = jnp.concatenate([offs_i + cnts_i, pad], axis=1)
    meta = jnp.where(row == 0, jnp.broadcast_to(bid, (8, 128)), 0)
    meta = jnp.where(row == 1, jnp.broadcast_to(eid, (8, 128)), meta)
    meta = jnp.where(row == 2, jnp.broadcast_to(offs_p, (8, 128)), meta)
    meta = jnp.where(row == 3, jnp.broadcast_to(ends_p, (8, 128)), meta)
    meta_ref[...] = meta


def _router(x, Wr1, br1, Wr2, br2):
    return pl.pallas_call(
        _router_body,
        out_shape=(
            jax.ShapeDtypeStruct((16, 128), jnp.int32),
            jax.ShapeDtypeStruct((8, 128), jnp.int32),
            jax.ShapeDtypeStruct((N, DW), jnp.uint32),
        ),
    )(x, Wr1, br1.reshape(1, DR), Wr2, br2.reshape(1, E))


def _dispatch_body(pos_hbm, xw_hbm, xws_hbm, idx_v, rows_v, sem):
    wid = lax.axis_index("s") * NC + lax.axis_index("c")
    base = wid * CHUNK
    pltpu.sync_copy(pos_hbm.at[pl.ds(base, CHUNK)], idx_v)
    pltpu.sync_copy(xw_hbm.at[pl.ds(base, CHUNK)], rows_v)
    pltpu.async_copy(rows_v, xws_hbm.at[idx_v], sem).wait()


_dispatch = pl.kernel(
    _dispatch_body,
    out_type=jax.ShapeDtypeStruct((N, DW), jnp.uint32),
    mesh=plsc.VectorSubcoreMesh(core_axis_name="c", subcore_axis_name="s"),
    scratch_types=[
        pltpu.VMEM((CHUNK,), jnp.int32),
        pltpu.VMEM((CHUNK, DW), jnp.uint32),
        pltpu.SemaphoreType.DMA,
    ],
)


def _mm_body(meta_ref, xs_ref, b1_ref, b2_ref, w1_hbm, w2_hbm, out_ref,
             w1_buf, w2_buf, sem1, sem2):
    def fetch(e, slot):
        pltpu.make_async_copy(w1_hbm.at[pl.ds(e, 1)], w1_buf.at[slot],
                              sem1.at[slot]).start()
        pltpu.make_async_copy(w2_hbm.at[pl.ds(e, 1)], w2_buf.at[slot],
                              sem2.at[slot]).start()

    def wait(e, slot):
        pltpu.make_async_copy(w1_hbm.at[pl.ds(e, 1)], w1_buf.at[slot],
                              sem1.at[slot]).wait()
        pltpu.make_async_copy(w2_hbm.at[pl.ds(e, 1)], w2_buf.at[slot],
                              sem2.at[slot]).wait()

    fetch(0, 0)
    for e in range(E):
        if e + 1 < E:
            fetch(e + 1, (e + 1) % 2)
        wait(e, e % 2)
        w1e = w1_buf[e % 2, 0].astype(jnp.bfloat16)
        w2e = w2_buf[e % 2, 0].astype(jnp.bfloat16)
        b1e = b1_ref[e]
        b2e = b2_ref[e]
        start = meta_ref[256 + e]
        end = meta_ref[384 + e]
        b_lo = lax.div(start, B)
        b_hi = jnp.where(end > start, lax.div(end - 1, B) + 1, b_lo)

        def blk(b, _):
            xw = xs_ref[pl.ds(b * B, B), :]
            px = xw[:, :XH]
            xa = lax.bitcast_convert_type(px << 16, jnp.float32)
            xb = lax.bitcast_convert_type(px & jnp.uint32(0xFFFF0000),
                                          jnp.float32)
            xbk = jnp.concatenate([xa, xb], axis=1).astype(jnp.bfloat16)
            wcol = lax.bitcast_convert_type(xw[:, XH:XH + 1], jnp.float32)
            h = jnp.maximum(
                jnp.dot(xbk, w1e, preferred_element_type=jnp.float32) + b1e,
                0.0)
            o = jnp.dot(h.astype(jnp.bfloat16), w2e,
                        preferred_element_type=jnp.float32) + b2e
            o = o * wcol
            rows = b * B + lax.broadcasted_iota(jnp.int32, (B, 1), 0)
            owned = (rows >= start) & (rows < end)
            out_ref[pl.ds(b * B, B), :] = jnp.where(
                owned, o, out_ref[pl.ds(b * B, B), :])
            return 0

        lax.fori_loop(b_lo, b_hi, blk, 0)


def _grouped_mm(meta_flat, xws, W1, b1, W2, b2):
    return pl.pallas_call(
        _mm_body,
        in_specs=[
            pl.BlockSpec(memory_space=pltpu.SMEM),
            pl.BlockSpec((N, DW), lambda: (0, 0),
                         pipeline_mode=pl.Buffered(buffer_count=1)),
            pl.BlockSpec((E, 1, F), lambda: (0, 0, 0),
                         pipeline_mode=pl.Buffered(buffer_count=1)),
            pl.BlockSpec((E, 1, O), lambda: (0, 0, 0),
                         pipeline_mode=pl.Buffered(buffer_count=1)),
            pl.BlockSpec(memory_space=pl.ANY),
            pl.BlockSpec(memory_space=pl.ANY),
        ],
        out_specs=pl.BlockSpec((N, O), lambda: (0, 0),
                               pipeline_mode=pl.Buffered(buffer_count=1)),
        out_shape=jax.ShapeDtypeStruct((N, O), jnp.float32),
        compiler_params=pltpu.CompilerParams(vmem_limit_bytes=100 * 1024 * 1024),
        scratch_shapes=[
            pltpu.VMEM((2, 1, D, F), jnp.float32),
            pltpu.VMEM((2, 1, F, O), jnp.float32),
            pltpu.SemaphoreType.DMA((2,)),
            pltpu.SemaphoreType.DMA((2,)),
        ],
    )(meta_flat, xws, b1.reshape(E, 1, F), b2.reshape(E, 1, O), W1, W2)


def _combine_body(pos_hbm, outs_hbm, out_hbm, idx_v, rows_v, sem):
    wid = lax.axis_index("s") * NC + lax.axis_index("c")
    base = wid * CHUNK
    pltpu.sync_copy(pos_hbm.at[pl.ds(base, CHUNK)], idx_v)
    pltpu.async_copy(outs_hbm.at[idx_v], rows_v, sem).wait()
    pltpu.sync_copy(rows_v, out_hbm.at[pl.ds(base, CHUNK)])


_combine = pl.kernel(
    _combine_body,
    out_type=jax.ShapeDtypeStruct((N, O), jnp.float32),
    mesh=plsc.VectorSubcoreMesh(core_axis_name="c", subcore_axis_name="s"),
    scratch_types=[
        pltpu.VMEM((CHUNK,), jnp.int32),
        pltpu.VMEM((CHUNK, O), jnp.float32),
        pltpu.SemaphoreType.DMA,
    ],
)


@jax.jit
def kernel(x, Wr1, br1, Wr2, br2, W1, b1, W2, b2):
    pos2d, meta, xw = _router(x, Wr1, br1, Wr2, br2)
    pos = pos2d.reshape(-1)
    xws = _dispatch(pos, xw)
    outs = _grouped_mm(meta.reshape(-1), xws, W1, b1, W2, b2)
    return _combine(pos, outs)
